# baseline (speedup 1.0000x reference)
import functools

import jax
import jax.numpy as jnp
from jax import lax
from jax.experimental import pallas as pl
from jax.experimental.pallas import tpu as pltpu
from jax.experimental.pallas import tpu_sc as plsc

_ROWS = 8192
_DIM = 1024

_info = plsc.get_sparse_core_info()
_NC = _info.num_cores
_RPC = _ROWS // _NC

_CH = 512
_NCHUNK = _RPC // _CH
_NBUF = 3


def _make_scs_copy():
    mesh = plsc.ScalarSubcoreMesh(axis_name="c", num_cores=_NC)

    @functools.partial(
        pl.kernel,
        mesh=mesh,
        out_type=jax.ShapeDtypeStruct((_ROWS, _DIM), jnp.float32),
        scratch_types=(
            [pltpu.VMEM_SHARED((_CH, _DIM), jnp.float32) for _ in range(_NBUF)]
            + [pltpu.SemaphoreType.DMA, pltpu.SemaphoreType.DMA]
        ),
    )
    def scs_copy(table_hbm, out_hbm, *scratch):
        bufs = scratch[:_NBUF]
        gsem, ssem = scratch[_NBUF], scratch[_NBUF + 1]
        base = lax.axis_index("c") * _RPC

        gathers = [None] * _NCHUNK
        scatters = [None] * _NCHUNK
        for i in range(_NCHUNK):
            b = bufs[i % _NBUF]
            if i >= _NBUF:
                scatters[i - _NBUF].wait()
            gathers[i] = pltpu.make_async_copy(
                table_hbm.at[pl.ds(base + i * _CH, _CH)], b, gsem
            )
            gathers[i].start()
            if i > 0:
                gathers[i - 1].wait()
                scatters[i - 1] = pltpu.make_async_copy(
                    bufs[(i - 1) % _NBUF],
                    out_hbm.at[pl.ds(base + (i - 1) * _CH, _CH)],
                    ssem,
                )
                scatters[i - 1].start()
        gathers[_NCHUNK - 1].wait()
        scatters[_NCHUNK - 1] = pltpu.make_async_copy(
            bufs[(_NCHUNK - 1) % _NBUF],
            out_hbm.at[pl.ds(base + (_NCHUNK - 1) * _CH, _CH)],
            ssem,
        )
        scatters[_NCHUNK - 1].start()
        for i in range(max(0, _NCHUNK - _NBUF), _NCHUNK):
            scatters[i].wait()

    return scs_copy


_scs_copy = _make_scs_copy()


@jax.jit
def kernel(x, pos_emb):
    del x
    return _scs_copy(pos_emb)
